# baseline (speedup 1.0000x reference)
import jax
import jax.numpy as jnp
from jax import lax
from jax.experimental import pallas as pl
from jax.experimental.pallas import tpu as pltpu
from jax.experimental.pallas import tpu_sc as plsc

N = 10000
B = 128
E = 320000
NC, NS, L = 2, 16, 16
W = NC * NS
BLK = 128
NBLK = E // BLK
BLK_PER_W = -(-NBLK // W)
STRIPE = 624
STRIPE_LAST = N - STRIPE * (NS - 1)
GRID = 10


def _exp_body(x_ref, o_ref):
    o_ref[...] = jnp.exp(x_ref[...])


def _log_body(p_ref, o_ref):
    s = p_ref[0] + p_ref[1]
    o_ref[...] = jnp.log(jnp.maximum(s, 1e-30))


def _sc_body(ev, meta, pf, zeros, out, md_v0, p_v0, rows_v0,
             md_v1, p_v1, rows_v1, s_sh, sem, ssem):
    cid = lax.axis_index("c")
    sid = lax.axis_index("s")
    wid = cid * NS + sid
    r0 = sid * STRIPE
    md_v = (md_v0, md_v1)
    p_v = (p_v0, p_v1)
    rows_v = (rows_v0, rows_v1)

    pltpu.sync_copy(meta.at[wid], md_v0)
    pltpu.sync_copy(pf.at[wid], p_v0)
    pltpu.async_copy(ev.at[md_v0.at[0]], rows_v0, sem)

    @pl.when(sid < NS - 1)
    def _():
        pltpu.sync_copy(zeros.at[pl.ds(r0, STRIPE)],
                        s_sh.at[pl.ds(r0, STRIPE)])

    @pl.when(sid == NS - 1)
    def _():
        pltpu.sync_copy(zeros.at[pl.ds(r0, STRIPE_LAST)],
                        s_sh.at[pl.ds(r0, STRIPE_LAST)])

    plsc.subcore_barrier()

    def do_pair(t2, carry):
        for u in range(2):
            t = 2 * t2 + u
            blk = wid + t * W
            mdc, pc, rc = md_v[u], p_v[u], rows_v[u]
            mdn, rn = md_v[1 - u], rows_v[1 - u]

            @pl.when(blk < NBLK)
            def _():
                pltpu.make_async_copy(ev.at[mdc.at[0]], rc, sem).wait()

                def wait_prev_scatter():
                    pltpu.make_async_copy(rn, s_sh.at[mdn.at[1]],
                                          ssem).wait()

                if u == 0:
                    pl.when(t2 >= 1)(wait_prev_scatter)
                else:
                    wait_prev_scatter()

                @pl.when(blk + W < NBLK)
                def _():
                    pltpu.sync_copy(meta.at[blk + W], mdn)
                    pltpu.sync_copy(pf.at[blk + W], p_v[1 - u])
                    pltpu.async_copy(ev.at[mdn.at[0]], rn, sem)

                def mul_group(g, c):
                    p16 = pc[0, pl.ds(g * L, L)]
                    for k in range(L):
                        ps = jnp.full((L,), p16[k], jnp.float32)
                        row = g * L + k
                        for j in range(B // L):
                            sl = (row, pl.ds(j * L, L))
                            rc[sl] = rc[sl] * ps
                    return c

                lax.fori_loop(0, BLK // L, mul_group, 0)

                pltpu.async_copy(rc, s_sh.at[mdc.at[1]], ssem, add=True)

        return carry

    lax.fori_loop(0, (BLK_PER_W + 1) // 2, do_pair, 0)
    pltpu.make_async_copy(rows_v0, s_sh.at[md_v0.at[1]], ssem).wait()
    plsc.subcore_barrier()

    @pl.when(sid < NS - 1)
    def _():
        pltpu.sync_copy(s_sh.at[pl.ds(r0, STRIPE)],
                        out.at[cid, pl.ds(r0, STRIPE)])

    @pl.when(sid == NS - 1)
    def _():
        pltpu.sync_copy(s_sh.at[pl.ds(r0, STRIPE_LAST)],
                        out.at[cid, pl.ds(r0, STRIPE_LAST)])


def kernel(ch_vals, edge_src, edge_dst, params):
    ev = pl.pallas_call(
        _exp_body,
        grid=(GRID,),
        in_specs=[pl.BlockSpec((N // GRID, B), lambda i: (i, 0))],
        out_specs=pl.BlockSpec((N // GRID, B), lambda i: (i, 0)),
        out_shape=jax.ShapeDtypeStruct((N, B), jnp.float32),
    )(ch_vals)

    meta = jnp.stack([edge_src.reshape(NBLK, BLK),
                      edge_dst.reshape(NBLK, BLK)], axis=1)
    pf = params.reshape(NBLK, 1, BLK)
    zeros = jnp.zeros((N, B), jnp.float32)

    sc = pl.kernel(
        _sc_body,
        out_type=jax.ShapeDtypeStruct((NC, N, B), jnp.float32),
        mesh=plsc.VectorSubcoreMesh(core_axis_name="c", subcore_axis_name="s"),
        scratch_types=[
            pltpu.VMEM((2, BLK), jnp.int32),
            pltpu.VMEM((1, BLK), jnp.float32),
            pltpu.VMEM((BLK, B), jnp.float32),
            pltpu.VMEM((2, BLK), jnp.int32),
            pltpu.VMEM((1, BLK), jnp.float32),
            pltpu.VMEM((BLK, B), jnp.float32),
            pltpu.VMEM_SHARED((N, B), jnp.float32),
            pltpu.SemaphoreType.DMA,
            pltpu.SemaphoreType.DMA,
        ],
    )
    partial = sc(ev, meta, pf, zeros)

    out = pl.pallas_call(
        _log_body,
        grid=(GRID,),
        in_specs=[pl.BlockSpec((NC, N // GRID, B), lambda i: (0, i, 0))],
        out_specs=pl.BlockSpec((N // GRID, B), lambda i: (i, 0)),
        out_shape=jax.ShapeDtypeStruct((N, B), jnp.float32),
    )(partial)
    return out
